# Initial kernel scaffold; baseline (speedup 1.0000x reference)
import jax
import jax.numpy as jnp
from jax.experimental import pallas as pl


def kernel(features, edge_index, W1, W2, W3):
    raise NotImplementedError("write your pallas kernel here")


# speedup vs baseline: 11.3136x; 11.3136x over previous
import functools

import jax
import jax.numpy as jnp
from jax import lax
from jax.experimental import pallas as pl
from jax.experimental.pallas import tpu as pltpu
from jax.experimental.pallas import tpu_sc as plsc

N_NODES = 10000
N_EDGES = 320000
D_FEAT = 128
L = 16
NPAD = 10240
NW = 32
EPW = N_EDGES // NW
CHUNK = 80
NCHUNK = EPW // CHUNK
ROWS_PER_TILE = NPAD // 16
ZROWS = 128


def _mm_body(x_ref, w_ref, o_ref):
    o_ref[...] = jnp.dot(x_ref[...], w_ref[...], preferred_element_type=jnp.float32)


def _node_matmul(features_pad, w1p):
    return pl.pallas_call(
        _mm_body,
        grid=(8,),
        in_specs=[
            pl.BlockSpec((NPAD // 8, D_FEAT), lambda i: (i, 0)),
            pl.BlockSpec((D_FEAT, L), lambda i: (0, 0)),
        ],
        out_specs=pl.BlockSpec((NPAD // 8, L), lambda i: (i, 0)),
        out_shape=jax.ShapeDtypeStruct((NPAD, L), jnp.float32),
    )(features_pad, w1p)


def _edge_body(g_hbm, src_hbm, dst_hbm, h1_out, cnt_out,
               src_v, dst_v, rows16_v, ones_v, zbuf, h1_sh, cnt_sh, sem):
    c = lax.axis_index("c")
    s = lax.axis_index("s")

    def _fill_z(i, carry):
        zbuf[i] = jnp.zeros((L,), jnp.float32)
        return carry
    lax.fori_loop(0, ZROWS, _fill_z, 0)

    def _fill_o(i, carry):
        ones_v[i] = jnp.ones((L,), jnp.float32)
        return carry
    lax.fori_loop(0, CHUNK, _fill_o, 0)

    row0 = s * ROWS_PER_TILE
    for j in range(ROWS_PER_TILE // ZROWS):
        pltpu.sync_copy(zbuf, h1_sh.at[pl.ds(row0 + j * ZROWS, ZROWS)])
        pltpu.sync_copy(zbuf, cnt_sh.at[pl.ds(row0 + j * ZROWS, ZROWS)])
    plsc.subcore_barrier()

    wid = s * 2 + c
    base = wid * EPW

    def _chunk(k, carry):
        off = base + k * CHUNK
        pltpu.sync_copy(src_hbm.at[pl.ds(off, CHUNK)], src_v)
        pltpu.sync_copy(dst_hbm.at[pl.ds(off, CHUNK)], dst_v)
        pltpu.async_copy(g_hbm.at[src_v], rows16_v, sem).wait()
        pltpu.sync_copy(rows16_v, h1_sh.at[dst_v], add=True)
        pltpu.sync_copy(ones_v, cnt_sh.at[src_v], add=True)
        return carry
    lax.fori_loop(0, NCHUNK, _chunk, 0)

    plsc.subcore_barrier()

    out_row = c * NPAD + row0
    pltpu.sync_copy(h1_sh.at[pl.ds(row0, ROWS_PER_TILE)],
                    h1_out.at[pl.ds(out_row, ROWS_PER_TILE)])
    pltpu.sync_copy(cnt_sh.at[pl.ds(row0, ROWS_PER_TILE)],
                    cnt_out.at[pl.ds(out_row, ROWS_PER_TILE)])


def _edge_pass(g, src, dst):
    mesh = plsc.VectorSubcoreMesh(core_axis_name="c", subcore_axis_name="s",
                                  num_cores=2, num_subcores=16)
    fn = functools.partial(
        pl.kernel,
        mesh=mesh,
        compiler_params=pltpu.CompilerParams(use_tc_tiling_on_sc=False),
        out_type=[
            jax.ShapeDtypeStruct((2 * NPAD, L), jnp.float32),
            jax.ShapeDtypeStruct((2 * NPAD, L), jnp.float32),
        ],
        scratch_types=[
            pltpu.VMEM((CHUNK,), jnp.int32),
            pltpu.VMEM((CHUNK,), jnp.int32),
            pltpu.VMEM((CHUNK, L), jnp.float32),
            pltpu.VMEM((CHUNK, L), jnp.float32),
            pltpu.VMEM((ZROWS, L), jnp.float32),
            pltpu.VMEM_SHARED((NPAD, L), jnp.float32),
            pltpu.VMEM_SHARED((NPAD, L), jnp.float32),
            pltpu.SemaphoreType.DMA,
        ],
    )(_edge_body)
    return fn(g, src, dst)


def _final_body(h_ref, c_ref, w2_ref, w3_ref, o_ref):
    h = h_ref[0:NPAD, :] + h_ref[NPAD:2 * NPAD, :]
    cnt = c_ref[0:NPAD, :] + c_ref[NPAD:2 * NPAD, :]
    z = jnp.maximum(h, 0.0) * cnt
    sm = jnp.sum(z, axis=0, keepdims=True) * (1.0 / N_NODES)
    v = jnp.dot(sm, w2_ref[...], preferred_element_type=jnp.float32)
    o = jnp.dot(v, w3_ref[...], preferred_element_type=jnp.float32)
    o_ref[...] = 1.0 / (1.0 + jnp.exp(-o))


def _final(h1p, cntp, w2p, w3p):
    return pl.pallas_call(
        _final_body,
        out_shape=jax.ShapeDtypeStruct((1, L), jnp.float32),
    )(h1p, cntp, w2p, w3p)


def kernel(features, edge_index, W1, W2, W3):
    src = edge_index[0].astype(jnp.int32)
    dst = edge_index[1].astype(jnp.int32)
    w1p = jnp.pad(W1.astype(jnp.float32), ((0, 0), (0, L - W1.shape[1])))
    w2p = jnp.pad(W2.astype(jnp.float32), ((0, L - W2.shape[0]), (0, L - W2.shape[1])))
    w3p = jnp.pad(W3.astype(jnp.float32), ((0, L - W3.shape[0]), (0, L - W3.shape[1])))
    fpad = jnp.pad(features.astype(jnp.float32), ((0, NPAD - N_NODES), (0, 0)))
    g = _node_matmul(fpad, w1p)
    h1p, cntp = _edge_pass(g, src, dst)
    full = _final(h1p, cntp, w2p, w3p)
    return full[:, 0:1]


# speedup vs baseline: 16.2445x; 1.4358x over previous
import functools

import jax
import jax.numpy as jnp
from jax import lax
from jax.experimental import pallas as pl
from jax.experimental.pallas import tpu as pltpu
from jax.experimental.pallas import tpu_sc as plsc

N_NODES = 10000
N_EDGES = 320000
D_FEAT = 128
L = 16
NPAD = 10240
NW = 32
EPW = N_EDGES // NW
CHUNK = 128
NCHUNK = EPW // CHUNK
TAIL = EPW - NCHUNK * CHUNK
ROWS_PER_TILE = NPAD // 16
ZROWS = 128


def _mm_body(x_ref, w_ref, o_ref):
    o_ref[...] = jnp.dot(x_ref[...], w_ref[...], preferred_element_type=jnp.float32)


def _node_matmul(features_pad, w1p):
    return pl.pallas_call(
        _mm_body,
        grid=(8,),
        in_specs=[
            pl.BlockSpec((NPAD // 8, D_FEAT), lambda i: (i, 0)),
            pl.BlockSpec((D_FEAT, L), lambda i: (0, 0)),
        ],
        out_specs=pl.BlockSpec((NPAD // 8, L), lambda i: (i, 0)),
        out_shape=jax.ShapeDtypeStruct((NPAD, L), jnp.float32),
    )(features_pad, w1p)


def _edge_body(g_hbm, src_hbm, dst_hbm, h1_out, cnt_out,
               src0, src1, dst0, dst1, rows0, rows1, srct, dstt, rowst,
               zbuf, cnt_priv, h1_sh,
               gsem0, gsem1, ssem0, ssem1):
    c = lax.axis_index("c")
    s = lax.axis_index("s")
    src_v = (src0, src1)
    dst_v = (dst0, dst1)
    rows_v = (rows0, rows1)
    gsem = (gsem0, gsem1)
    ssem = (ssem0, ssem1)

    def _fill_c(i, carry):
        cnt_priv[pl.ds(i * L, L)] = jnp.zeros((L,), jnp.float32)
        return carry
    lax.fori_loop(0, NPAD // L, _fill_c, 0)

    def _fill_z(i, carry):
        zbuf[i] = jnp.zeros((L,), jnp.float32)
        return carry
    lax.fori_loop(0, ZROWS, _fill_z, 0)
    row0 = s * ROWS_PER_TILE
    for j in range(ROWS_PER_TILE // ZROWS):
        pltpu.sync_copy(zbuf, h1_sh.at[pl.ds(row0 + j * ZROWS, ZROWS)])
    plsc.subcore_barrier()

    wid = s * 2 + c
    base = wid * EPW

    def _load_idx(k, slot):
        off = base + k * CHUNK
        pltpu.sync_copy(src_hbm.at[pl.ds(off, CHUNK)], src_v[slot])
        pltpu.sync_copy(dst_hbm.at[pl.ds(off, CHUNK)], dst_v[slot])

    def _fire_gather(slot):
        pltpu.async_copy(g_hbm.at[src_v[slot]], rows_v[slot], gsem[slot])

    def _wait_gather(slot):
        pltpu.make_async_copy(g_hbm.at[src_v[slot]], rows_v[slot],
                              gsem[slot]).wait()

    def _fire_scatter(slot):
        pltpu.async_copy(rows_v[slot], h1_sh.at[dst_v[slot]], ssem[slot],
                         add=True)

    def _wait_scatter(slot):
        pltpu.make_async_copy(rows_v[slot], h1_sh.at[dst_v[slot]],
                              ssem[slot]).wait()

    def _histogram(slot):
        ones16 = jnp.ones((L,), jnp.float32)
        for j in range(CHUNK // L):
            idx = src_v[slot][pl.ds(j * L, L)]
            plsc.addupdate_scatter(cnt_priv, [idx], ones16)

    _load_idx(0, 0)
    _fire_gather(0)

    def _half(k, q, slot):
        _wait_gather(slot)
        _histogram(slot)
        _fire_scatter(slot)
        def _prefetch():
            @pl.when(k >= 1)
            def _():
                _wait_scatter(1 - slot)
            _load_idx(k + 1, 1 - slot)
            _fire_gather(1 - slot)
        if slot == 0:
            _prefetch()
        else:
            pl.when(q < NCHUNK // 2 - 1)(_prefetch)

    def _pair(q, carry):
        _half(2 * q, q, 0)
        _half(2 * q + 1, q, 1)
        return carry
    lax.fori_loop(0, NCHUNK // 2, _pair, 0)

    _wait_scatter(0)
    _wait_scatter(1)

    off = base + NCHUNK * CHUNK
    pltpu.sync_copy(src_hbm.at[pl.ds(off, TAIL)], srct)
    pltpu.sync_copy(dst_hbm.at[pl.ds(off, TAIL)], dstt)
    pltpu.async_copy(g_hbm.at[srct], rowst, gsem0).wait()
    plsc.addupdate_scatter(cnt_priv, [srct[...]], jnp.ones((L,), jnp.float32))
    pltpu.sync_copy(rowst, h1_sh.at[dstt], add=True)

    plsc.subcore_barrier()

    out_row = c * NPAD + row0
    pltpu.sync_copy(h1_sh.at[pl.ds(row0, ROWS_PER_TILE)],
                    h1_out.at[pl.ds(out_row, ROWS_PER_TILE)])
    pltpu.sync_copy(cnt_priv, cnt_out.at[wid])


def _edge_pass(g, src, dst):
    mesh = plsc.VectorSubcoreMesh(core_axis_name="c", subcore_axis_name="s",
                                  num_cores=2, num_subcores=16)
    fn = functools.partial(
        pl.kernel,
        mesh=mesh,
        compiler_params=pltpu.CompilerParams(use_tc_tiling_on_sc=False,
                                             needs_layout_passes=False),
        out_type=[
            jax.ShapeDtypeStruct((2 * NPAD, L), jnp.float32),
            jax.ShapeDtypeStruct((NW, NPAD), jnp.float32),
        ],
        scratch_types=[
            pltpu.VMEM((CHUNK,), jnp.int32),
            pltpu.VMEM((CHUNK,), jnp.int32),
            pltpu.VMEM((CHUNK,), jnp.int32),
            pltpu.VMEM((CHUNK,), jnp.int32),
            pltpu.VMEM((CHUNK, L), jnp.float32),
            pltpu.VMEM((CHUNK, L), jnp.float32),
            pltpu.VMEM((TAIL,), jnp.int32),
            pltpu.VMEM((TAIL,), jnp.int32),
            pltpu.VMEM((TAIL, L), jnp.float32),
            pltpu.VMEM((ZROWS, L), jnp.float32),
            pltpu.VMEM((NPAD,), jnp.float32),
            pltpu.VMEM_SHARED((NPAD, L), jnp.float32),
            pltpu.SemaphoreType.DMA,
            pltpu.SemaphoreType.DMA,
            pltpu.SemaphoreType.DMA,
            pltpu.SemaphoreType.DMA,
        ],
    )(_edge_body)
    return fn(g, src, dst)


def _final_body(h_ref, c_ref, w2_ref, w3_ref, o_ref):
    h = h_ref[0:NPAD, :] + h_ref[NPAD:2 * NPAD, :]
    cnt_row = jnp.sum(c_ref[...], axis=0, keepdims=True)
    z = jnp.maximum(h, 0.0)
    sm = jnp.dot(cnt_row, z, preferred_element_type=jnp.float32) * (1.0 / N_NODES)
    v = jnp.dot(sm, w2_ref[...], preferred_element_type=jnp.float32)
    o = jnp.dot(v, w3_ref[...], preferred_element_type=jnp.float32)
    o_ref[...] = 1.0 / (1.0 + jnp.exp(-o))


def _final(h1p, cntp, w2p, w3p):
    return pl.pallas_call(
        _final_body,
        out_shape=jax.ShapeDtypeStruct((1, L), jnp.float32),
    )(h1p, cntp, w2p, w3p)


def kernel(features, edge_index, W1, W2, W3):
    src = edge_index[0].astype(jnp.int32)
    dst = edge_index[1].astype(jnp.int32)
    w1p = jnp.pad(W1.astype(jnp.float32), ((0, 0), (0, L - W1.shape[1])))
    w2p = jnp.pad(W2.astype(jnp.float32), ((0, L - W2.shape[0]), (0, L - W2.shape[1])))
    w3p = jnp.pad(W3.astype(jnp.float32), ((0, L - W3.shape[0]), (0, L - W3.shape[1])))
    fpad = jnp.pad(features.astype(jnp.float32), ((0, NPAD - N_NODES), (0, 0)))
    g = _node_matmul(fpad, w1p)
    h1p, cntp = _edge_pass(g, src, dst)
    full = _final(h1p, cntp, w2p, w3p)
    return full[:, 0:1]


# speedup vs baseline: 16.4095x; 1.0102x over previous
import functools

import jax
import jax.numpy as jnp
from jax import lax
from jax.experimental import pallas as pl
from jax.experimental.pallas import tpu as pltpu
from jax.experimental.pallas import tpu_sc as plsc

N_NODES = 10000
N_EDGES = 320000
D_FEAT = 128
L = 16
NPAD = 10240
NW = 32
EPW = N_EDGES // NW
CHUNK = 128
NCHUNK = EPW // CHUNK
TAIL = EPW - NCHUNK * CHUNK
ROWS_PER_TILE = NPAD // 16
ZROWS = 128


def _mm_body(x_ref, w_ref, o_ref):
    o_ref[...] = jnp.dot(x_ref[...], w_ref[...],
                         preferred_element_type=jnp.float32).astype(jnp.bfloat16)


def _node_matmul(features_pad, w1p):
    return pl.pallas_call(
        _mm_body,
        grid=(8,),
        in_specs=[
            pl.BlockSpec((NPAD // 8, D_FEAT), lambda i: (i, 0)),
            pl.BlockSpec((D_FEAT, L), lambda i: (0, 0)),
        ],
        out_specs=pl.BlockSpec((NPAD // 8, L), lambda i: (i, 0)),
        out_shape=jax.ShapeDtypeStruct((NPAD, L), jnp.bfloat16),
    )(features_pad, w1p)


def _edge_body(g_hbm, src_hbm, dst_hbm, h1_out, cnt_out,
               src0, src1, dst0, dst1, rows0, rows1, srct, dstt, rowst,
               zbuf, cnt_priv, h1_sh,
               gsem0, gsem1, ssem0, ssem1):
    c = lax.axis_index("c")
    s = lax.axis_index("s")
    src_v = (src0, src1)
    dst_v = (dst0, dst1)
    rows_v = (rows0, rows1)
    gsem = (gsem0, gsem1)
    ssem = (ssem0, ssem1)

    def _fill_c(i, carry):
        cnt_priv[pl.ds(i * L, L)] = jnp.zeros((L,), jnp.float32)
        return carry
    lax.fori_loop(0, NPAD // L, _fill_c, 0)

    def _fill_z(i, carry):
        zbuf[pl.ds(2 * i, 2), :] = jnp.zeros((2, L), jnp.bfloat16)
        return carry
    lax.fori_loop(0, ZROWS // 2, _fill_z, 0)
    row0 = s * ROWS_PER_TILE
    for j in range(ROWS_PER_TILE // ZROWS):
        pltpu.sync_copy(zbuf, h1_sh.at[pl.ds(row0 + j * ZROWS, ZROWS)])
    plsc.subcore_barrier()

    wid = s * 2 + c
    base = wid * EPW

    def _load_idx(k, slot):
        off = base + k * CHUNK
        pltpu.sync_copy(src_hbm.at[pl.ds(off, CHUNK)], src_v[slot])
        pltpu.sync_copy(dst_hbm.at[pl.ds(off, CHUNK)], dst_v[slot])

    def _fire_gather(slot):
        pltpu.async_copy(g_hbm.at[src_v[slot]], rows_v[slot], gsem[slot])

    def _wait_gather(slot):
        pltpu.make_async_copy(g_hbm.at[src_v[slot]], rows_v[slot],
                              gsem[slot]).wait()

    def _fire_scatter(slot):
        pltpu.async_copy(rows_v[slot], h1_sh.at[dst_v[slot]], ssem[slot],
                         add=True)

    def _wait_scatter(slot):
        pltpu.make_async_copy(rows_v[slot], h1_sh.at[dst_v[slot]],
                              ssem[slot]).wait()

    def _histogram(slot):
        ones16 = jnp.ones((L,), jnp.float32)
        for j in range(CHUNK // L):
            idx = src_v[slot][pl.ds(j * L, L)]
            plsc.addupdate_scatter(cnt_priv, [idx], ones16)

    _load_idx(0, 0)
    _fire_gather(0)

    def _half(k, q, slot):
        _wait_gather(slot)
        _histogram(slot)
        _fire_scatter(slot)
        def _prefetch():
            @pl.when(k >= 1)
            def _():
                _wait_scatter(1 - slot)
            _load_idx(k + 1, 1 - slot)
            _fire_gather(1 - slot)
        if slot == 0:
            _prefetch()
        else:
            pl.when(q < NCHUNK // 2 - 1)(_prefetch)

    def _pair(q, carry):
        _half(2 * q, q, 0)
        _half(2 * q + 1, q, 1)
        return carry
    lax.fori_loop(0, NCHUNK // 2, _pair, 0)

    _wait_scatter(0)
    _wait_scatter(1)

    off = base + NCHUNK * CHUNK
    pltpu.sync_copy(src_hbm.at[pl.ds(off, TAIL)], srct)
    pltpu.sync_copy(dst_hbm.at[pl.ds(off, TAIL)], dstt)
    pltpu.async_copy(g_hbm.at[srct], rowst, gsem0).wait()
    plsc.addupdate_scatter(cnt_priv, [srct[...]], jnp.ones((L,), jnp.float32))
    pltpu.sync_copy(rowst, h1_sh.at[dstt], add=True)

    plsc.subcore_barrier()

    out_row = c * NPAD + row0
    pltpu.sync_copy(h1_sh.at[pl.ds(row0, ROWS_PER_TILE)],
                    h1_out.at[pl.ds(out_row, ROWS_PER_TILE)])
    pltpu.sync_copy(cnt_priv, cnt_out.at[wid])


def _edge_pass(g, src, dst):
    mesh = plsc.VectorSubcoreMesh(core_axis_name="c", subcore_axis_name="s",
                                  num_cores=2, num_subcores=16)
    fn = functools.partial(
        pl.kernel,
        mesh=mesh,
        compiler_params=pltpu.CompilerParams(use_tc_tiling_on_sc=False,
                                             needs_layout_passes=False),
        out_type=[
            jax.ShapeDtypeStruct((2 * NPAD, L), jnp.bfloat16),
            jax.ShapeDtypeStruct((NW, NPAD), jnp.float32),
        ],
        scratch_types=[
            pltpu.VMEM((CHUNK,), jnp.int32),
            pltpu.VMEM((CHUNK,), jnp.int32),
            pltpu.VMEM((CHUNK,), jnp.int32),
            pltpu.VMEM((CHUNK,), jnp.int32),
            pltpu.VMEM((CHUNK, L), jnp.bfloat16),
            pltpu.VMEM((CHUNK, L), jnp.bfloat16),
            pltpu.VMEM((TAIL,), jnp.int32),
            pltpu.VMEM((TAIL,), jnp.int32),
            pltpu.VMEM((TAIL, L), jnp.bfloat16),
            pltpu.VMEM((ZROWS, L), jnp.bfloat16),
            pltpu.VMEM((NPAD,), jnp.float32),
            pltpu.VMEM_SHARED((NPAD, L), jnp.bfloat16),
            pltpu.SemaphoreType.DMA,
            pltpu.SemaphoreType.DMA,
            pltpu.SemaphoreType.DMA,
            pltpu.SemaphoreType.DMA,
        ],
    )(_edge_body)
    return fn(g, src, dst)


def _final_body(h_ref, c_ref, w2_ref, w3_ref, o_ref):
    h = (h_ref[0:NPAD, :].astype(jnp.float32)
         + h_ref[NPAD:2 * NPAD, :].astype(jnp.float32))
    cnt_row = jnp.sum(c_ref[...], axis=0, keepdims=True)
    z = jnp.maximum(h, 0.0)
    sm = jnp.dot(cnt_row, z, preferred_element_type=jnp.float32) * (1.0 / N_NODES)
    v = jnp.dot(sm, w2_ref[...], preferred_element_type=jnp.float32)
    o = jnp.dot(v, w3_ref[...], preferred_element_type=jnp.float32)
    o_ref[...] = 1.0 / (1.0 + jnp.exp(-o))


def _final(h1p, cntp, w2p, w3p):
    return pl.pallas_call(
        _final_body,
        out_shape=jax.ShapeDtypeStruct((1, L), jnp.float32),
    )(h1p, cntp, w2p, w3p)


def kernel(features, edge_index, W1, W2, W3):
    src = edge_index[0].astype(jnp.int32)
    dst = edge_index[1].astype(jnp.int32)
    w1p = jnp.pad(W1.astype(jnp.float32), ((0, 0), (0, L - W1.shape[1])))
    w2p = jnp.pad(W2.astype(jnp.float32), ((0, L - W2.shape[0]), (0, L - W2.shape[1])))
    w3p = jnp.pad(W3.astype(jnp.float32), ((0, L - W3.shape[0]), (0, L - W3.shape[1])))
    fpad = jnp.pad(features.astype(jnp.float32), ((0, NPAD - N_NODES), (0, 0)))
    g = _node_matmul(fpad, w1p)
    h1p, cntp = _edge_pass(g, src, dst)
    full = _final(h1p, cntp, w2p, w3p)
    return full[:, 0:1]


# speedup vs baseline: 24.9623x; 1.5212x over previous
import functools

import jax
import jax.numpy as jnp
from jax import lax
from jax.experimental import pallas as pl
from jax.experimental.pallas import tpu as pltpu
from jax.experimental.pallas import tpu_sc as plsc

N_NODES = 10000
N_EDGES = 320000
D_FEAT = 128
L = 16
NPAD = 10240
NW = 32
EPW = N_EDGES // NW
CHUNK = 128
NCHUNK = EPW // CHUNK
TAIL = EPW - NCHUNK * CHUNK
ROWS_PER_TILE = NPAD // 16
ZROWS = 128


def _mm_body(x_ref, w_ref, o_ref):
    o_ref[...] = jnp.dot(x_ref[...], w_ref[...],
                         preferred_element_type=jnp.float32).astype(jnp.bfloat16)


def _node_matmul(features_pad, w1p):
    return pl.pallas_call(
        _mm_body,
        grid=(8,),
        in_specs=[
            pl.BlockSpec((NPAD // 8, D_FEAT), lambda i: (i, 0)),
            pl.BlockSpec((D_FEAT, L), lambda i: (0, 0)),
        ],
        out_specs=pl.BlockSpec((NPAD // 8, L), lambda i: (i, 0)),
        out_shape=jax.ShapeDtypeStruct((NPAD, L), jnp.bfloat16),
    )(features_pad, w1p)


def _edge_body(g_hbm, src_hbm, dst_hbm, h1_out, cnt_out,
               src_all, dst_all, rows0, rows1, rowst,
               zbuf, cnt_priv, h1_sh,
               gsem0, gsem1, ssem0, ssem1):
    c = lax.axis_index("c")
    s = lax.axis_index("s")
    rows_v = (rows0, rows1)
    gsem = (gsem0, gsem1)
    ssem = (ssem0, ssem1)

    def _fill_c(i, carry):
        cnt_priv[pl.ds(i * L, L)] = jnp.zeros((L,), jnp.float32)
        return carry
    lax.fori_loop(0, NPAD // L, _fill_c, 0)

    def _fill_z(i, carry):
        zbuf[pl.ds(2 * i, 2), :] = jnp.zeros((2, L), jnp.bfloat16)
        return carry
    lax.fori_loop(0, ZROWS // 2, _fill_z, 0)
    row0 = s * ROWS_PER_TILE
    for j in range(ROWS_PER_TILE // ZROWS):
        pltpu.sync_copy(zbuf, h1_sh.at[pl.ds(row0 + j * ZROWS, ZROWS)])
    plsc.subcore_barrier()

    wid = s * 2 + c
    base = wid * EPW
    pltpu.sync_copy(src_hbm.at[pl.ds(base, EPW)], src_all)
    pltpu.sync_copy(dst_hbm.at[pl.ds(base, EPW)], dst_all)

    def _src_idx(k, n=CHUNK):
        return src_all.at[pl.ds(k * CHUNK, n)]

    def _dst_idx(k, n=CHUNK):
        return dst_all.at[pl.ds(k * CHUNK, n)]

    def _fire_gather(k, slot):
        pltpu.async_copy(g_hbm.at[_src_idx(k)], rows_v[slot], gsem[slot])

    def _wait_gather(k, slot):
        pltpu.make_async_copy(g_hbm.at[_src_idx(k)], rows_v[slot],
                              gsem[slot]).wait()

    def _fire_scatter(k, slot):
        pltpu.async_copy(rows_v[slot], h1_sh.at[_dst_idx(k)], ssem[slot],
                         add=True)

    def _wait_scatter(k, slot):
        pltpu.make_async_copy(rows_v[slot], h1_sh.at[_dst_idx(k)],
                              ssem[slot]).wait()

    iota16 = lax.iota(jnp.int32, L)
    ones16 = jnp.ones((L,), jnp.float32)

    def _histogram(k):
        for j in range(CHUNK // L):
            pos = iota16 + (k * CHUNK + j * L)
            ids = plsc.load_gather(src_all, [pos])
            plsc.addupdate_scatter(cnt_priv, [ids], ones16)

    _fire_gather(0, 0)

    def _half(k, q, slot):
        _wait_gather(k, slot)
        _histogram(k)
        _fire_scatter(k, slot)
        def _prefetch():
            @pl.when(k >= 1)
            def _():
                _wait_scatter(k - 1, 1 - slot)
            _fire_gather(k + 1, 1 - slot)
        if slot == 0:
            _prefetch()
        else:
            pl.when(q < NCHUNK // 2 - 1)(_prefetch)

    def _pair(q, carry):
        _half(2 * q, q, 0)
        _half(2 * q + 1, q, 1)
        return carry
    lax.fori_loop(0, NCHUNK // 2, _pair, 0)

    _wait_scatter(NCHUNK - 2, 0)
    _wait_scatter(NCHUNK - 1, 1)

    pltpu.async_copy(g_hbm.at[_src_idx(NCHUNK, TAIL)], rowst, gsem0).wait()
    for j in range(TAIL // L):
        pos = iota16 + (NCHUNK * CHUNK + j * L)
        ids = plsc.load_gather(src_all, [pos])
        plsc.addupdate_scatter(cnt_priv, [ids], ones16)
    pltpu.sync_copy(rowst, h1_sh.at[_dst_idx(NCHUNK, TAIL)], add=True)

    plsc.subcore_barrier()

    out_row = c * NPAD + row0
    pltpu.sync_copy(h1_sh.at[pl.ds(row0, ROWS_PER_TILE)],
                    h1_out.at[pl.ds(out_row, ROWS_PER_TILE)])
    pltpu.sync_copy(cnt_priv, cnt_out.at[wid])


def _edge_pass(g, src, dst):
    mesh = plsc.VectorSubcoreMesh(core_axis_name="c", subcore_axis_name="s",
                                  num_cores=2, num_subcores=16)
    fn = functools.partial(
        pl.kernel,
        mesh=mesh,
        compiler_params=pltpu.CompilerParams(use_tc_tiling_on_sc=False,
                                             needs_layout_passes=False),
        out_type=[
            jax.ShapeDtypeStruct((2 * NPAD, L), jnp.bfloat16),
            jax.ShapeDtypeStruct((NW, NPAD), jnp.float32),
        ],
        scratch_types=[
            pltpu.VMEM((EPW,), jnp.int32),
            pltpu.VMEM((EPW,), jnp.int32),
            pltpu.VMEM((CHUNK, L), jnp.bfloat16),
            pltpu.VMEM((CHUNK, L), jnp.bfloat16),
            pltpu.VMEM((TAIL, L), jnp.bfloat16),
            pltpu.VMEM((ZROWS, L), jnp.bfloat16),
            pltpu.VMEM((NPAD,), jnp.float32),
            pltpu.VMEM_SHARED((NPAD, L), jnp.bfloat16),
            pltpu.SemaphoreType.DMA,
            pltpu.SemaphoreType.DMA,
            pltpu.SemaphoreType.DMA,
            pltpu.SemaphoreType.DMA,
        ],
    )(_edge_body)
    return fn(g, src, dst)


def _final_body(h_ref, c_ref, w2_ref, w3_ref, o_ref):
    h = (h_ref[0:NPAD, :].astype(jnp.float32)
         + h_ref[NPAD:2 * NPAD, :].astype(jnp.float32))
    cnt_row = jnp.sum(c_ref[...], axis=0, keepdims=True)
    z = jnp.maximum(h, 0.0)
    sm = jnp.dot(cnt_row, z, preferred_element_type=jnp.float32) * (1.0 / N_NODES)
    v = jnp.dot(sm, w2_ref[...], preferred_element_type=jnp.float32)
    o = jnp.dot(v, w3_ref[...], preferred_element_type=jnp.float32)
    o_ref[...] = 1.0 / (1.0 + jnp.exp(-o))


def _final(h1p, cntp, w2p, w3p):
    return pl.pallas_call(
        _final_body,
        out_shape=jax.ShapeDtypeStruct((1, L), jnp.float32),
    )(h1p, cntp, w2p, w3p)


def kernel(features, edge_index, W1, W2, W3):
    src = edge_index[0].astype(jnp.int32)
    dst = edge_index[1].astype(jnp.int32)
    w1p = jnp.pad(W1.astype(jnp.float32), ((0, 0), (0, L - W1.shape[1])))
    w2p = jnp.pad(W2.astype(jnp.float32), ((0, L - W2.shape[0]), (0, L - W2.shape[1])))
    w3p = jnp.pad(W3.astype(jnp.float32), ((0, L - W3.shape[0]), (0, L - W3.shape[1])))
    fpad = jnp.pad(features.astype(jnp.float32), ((0, NPAD - N_NODES), (0, 0)))
    g = _node_matmul(fpad, w1p)
    h1p, cntp = _edge_pass(g, src, dst)
    full = _final(h1p, cntp, w2p, w3p)
    return full[:, 0:1]


# speedup vs baseline: 32.9110x; 1.3184x over previous
import functools

import jax
import jax.numpy as jnp
from jax import lax
from jax.experimental import pallas as pl
from jax.experimental.pallas import tpu as pltpu
from jax.experimental.pallas import tpu_sc as plsc

N_NODES = 10000
N_EDGES = 320000
D_FEAT = 128
L = 16
NPAD = 10240
NW = 32
EPW = N_EDGES // NW
CHUNK = 128
NCHUNK = EPW // CHUNK
TAIL = EPW - NCHUNK * CHUNK
ROWS_PER_TILE = NPAD // 16
ZROWS = 128


def _mm_body(x_ref, w_ref, o_ref):
    o_ref[...] = jnp.dot(x_ref[...], w_ref[...],
                         preferred_element_type=jnp.float32).astype(jnp.bfloat16)


def _node_matmul(features, w1p):
    return pl.pallas_call(
        _mm_body,
        grid=(10,),
        in_specs=[
            pl.BlockSpec((N_NODES // 10, D_FEAT), lambda i: (i, 0)),
            pl.BlockSpec((D_FEAT, L), lambda i: (0, 0)),
        ],
        out_specs=pl.BlockSpec((N_NODES // 10, L), lambda i: (i, 0)),
        out_shape=jax.ShapeDtypeStruct((N_NODES, L), jnp.bfloat16),
    )(features, w1p)


def _edge_body(g_hbm, src_hbm, dst_hbm, h1_out, cnt_out,
               src_all, dst_all, rows0, rows1, rows2, rows3, rowst,
               zbuf, cnt_priv, h1_sh,
               gsem0, gsem1, gsem2, gsem3, ssem0, ssem1, ssem2, ssem3):
    c = lax.axis_index("c")
    s = lax.axis_index("s")
    rows_v = (rows0, rows1, rows2, rows3)
    gsem = (gsem0, gsem1, gsem2, gsem3)
    ssem = (ssem0, ssem1, ssem2, ssem3)

    def _fill_c(i, carry):
        cnt_priv[pl.ds(i * L, L)] = jnp.zeros((L,), jnp.float32)
        return carry
    lax.fori_loop(0, NPAD // L, _fill_c, 0)

    def _fill_z(i, carry):
        zbuf[pl.ds(2 * i, 2), :] = jnp.zeros((2, L), jnp.bfloat16)
        return carry
    lax.fori_loop(0, ZROWS // 2, _fill_z, 0)
    row0 = s * ROWS_PER_TILE
    for j in range(ROWS_PER_TILE // ZROWS):
        pltpu.sync_copy(zbuf, h1_sh.at[pl.ds(row0 + j * ZROWS, ZROWS)])
    plsc.subcore_barrier()

    wid = s * 2 + c
    base = wid * EPW
    pltpu.sync_copy(src_hbm.at[pl.ds(base, EPW)], src_all)
    pltpu.sync_copy(dst_hbm.at[pl.ds(base, EPW)], dst_all)

    def _src_idx(k, n=CHUNK):
        return src_all.at[pl.ds(k * CHUNK, n)]

    def _dst_idx(k, n=CHUNK):
        return dst_all.at[pl.ds(k * CHUNK, n)]

    def _fire_gather(k, slot):
        pltpu.async_copy(g_hbm.at[_src_idx(k)], rows_v[slot], gsem[slot])

    def _wait_gather(k, slot):
        pltpu.make_async_copy(g_hbm.at[_src_idx(k)], rows_v[slot],
                              gsem[slot]).wait()

    def _fire_scatter(k, slot):
        pltpu.async_copy(rows_v[slot], h1_sh.at[_dst_idx(k)], ssem[slot],
                         add=True)

    def _wait_scatter(k, slot):
        pltpu.make_async_copy(rows_v[slot], h1_sh.at[_dst_idx(k)],
                              ssem[slot]).wait()

    iota16 = lax.iota(jnp.int32, L)
    ones16 = jnp.ones((L,), jnp.float32)

    def _histogram(k):
        for j in range(CHUNK // L):
            pos = iota16 + (k * CHUNK + j * L)
            ids = plsc.load_gather(src_all, [pos])
            plsc.addupdate_scatter(cnt_priv, [ids], ones16)

    _fire_gather(0, 0)
    _fire_gather(1, 1)

    def _step(k, slot):
        _wait_gather(k, slot)
        _histogram(k)
        _fire_scatter(k, slot)
        nslot = (slot + 2) % 4

        @pl.when(k >= 2)
        def _():
            _wait_scatter(k - 2, nslot)
        _fire_gather(k + 2, nslot)

    def _quad(q, carry):
        for r in range(4):
            _step(4 * q + r, r)
        return carry
    lax.fori_loop(0, NCHUNK // 4, _quad, 0)

    for k in (NCHUNK - 2, NCHUNK - 1):
        slot = k % 4
        _wait_gather(k, slot)
        _histogram(k)
        _fire_scatter(k, slot)
        _wait_scatter(k - 2, (slot + 2) % 4)
    _wait_scatter(NCHUNK - 2, (NCHUNK - 2) % 4)
    _wait_scatter(NCHUNK - 1, (NCHUNK - 1) % 4)

    pltpu.async_copy(g_hbm.at[_src_idx(NCHUNK, TAIL)], rowst, gsem0).wait()
    for j in range(TAIL // L):
        pos = iota16 + (NCHUNK * CHUNK + j * L)
        ids = plsc.load_gather(src_all, [pos])
        plsc.addupdate_scatter(cnt_priv, [ids], ones16)
    pltpu.sync_copy(rowst, h1_sh.at[_dst_idx(NCHUNK, TAIL)], add=True)

    plsc.subcore_barrier()

    out_row = c * NPAD + row0
    pltpu.sync_copy(h1_sh.at[pl.ds(row0, ROWS_PER_TILE)],
                    h1_out.at[pl.ds(out_row, ROWS_PER_TILE)])
    pltpu.sync_copy(cnt_priv, cnt_out.at[wid])


def _edge_pass(g, src, dst):
    mesh = plsc.VectorSubcoreMesh(core_axis_name="c", subcore_axis_name="s",
                                  num_cores=2, num_subcores=16)
    fn = functools.partial(
        pl.kernel,
        mesh=mesh,
        compiler_params=pltpu.CompilerParams(use_tc_tiling_on_sc=False,
                                             needs_layout_passes=False),
        out_type=[
            jax.ShapeDtypeStruct((2 * NPAD, L), jnp.bfloat16),
            jax.ShapeDtypeStruct((NW, NPAD), jnp.float32),
        ],
        scratch_types=[
            pltpu.VMEM((EPW,), jnp.int32),
            pltpu.VMEM((EPW,), jnp.int32),
            pltpu.VMEM((CHUNK, L), jnp.bfloat16),
            pltpu.VMEM((CHUNK, L), jnp.bfloat16),
            pltpu.VMEM((CHUNK, L), jnp.bfloat16),
            pltpu.VMEM((CHUNK, L), jnp.bfloat16),
            pltpu.VMEM((TAIL, L), jnp.bfloat16),
            pltpu.VMEM((ZROWS, L), jnp.bfloat16),
            pltpu.VMEM((NPAD,), jnp.float32),
            pltpu.VMEM_SHARED((NPAD, L), jnp.bfloat16),
            pltpu.SemaphoreType.DMA,
            pltpu.SemaphoreType.DMA,
            pltpu.SemaphoreType.DMA,
            pltpu.SemaphoreType.DMA,
            pltpu.SemaphoreType.DMA,
            pltpu.SemaphoreType.DMA,
            pltpu.SemaphoreType.DMA,
            pltpu.SemaphoreType.DMA,
        ],
    )(_edge_body)
    return fn(g, src, dst)


def _final_body(h_ref, c_ref, w2_ref, w3_ref, o_ref):
    h = (h_ref[0:NPAD, :].astype(jnp.float32)
         + h_ref[NPAD:2 * NPAD, :].astype(jnp.float32))
    cnt_row = jnp.sum(c_ref[...], axis=0, keepdims=True)
    z = jnp.maximum(h, 0.0)
    sm = jnp.dot(cnt_row, z, preferred_element_type=jnp.float32) * (1.0 / N_NODES)
    v = jnp.dot(sm, w2_ref[...], preferred_element_type=jnp.float32)
    o = jnp.dot(v, w3_ref[...], preferred_element_type=jnp.float32)
    o_ref[...] = 1.0 / (1.0 + jnp.exp(-o))


def _final(h1p, cntp, w2p, w3p):
    return pl.pallas_call(
        _final_body,
        out_shape=jax.ShapeDtypeStruct((1, L), jnp.float32),
    )(h1p, cntp, w2p, w3p)


def kernel(features, edge_index, W1, W2, W3):
    src = edge_index[0].astype(jnp.int32)
    dst = edge_index[1].astype(jnp.int32)
    w1p = jnp.pad(W1.astype(jnp.float32), ((0, 0), (0, L - W1.shape[1])))
    w2p = jnp.pad(W2.astype(jnp.float32), ((0, L - W2.shape[0]), (0, L - W2.shape[1])))
    w3p = jnp.pad(W3.astype(jnp.float32), ((0, L - W3.shape[0]), (0, L - W3.shape[1])))
    g = _node_matmul(features.astype(jnp.float32), w1p)
    h1p, cntp = _edge_pass(g, src, dst)
    full = _final(h1p, cntp, w2p, w3p)
    return full[:, 0:1]


# speedup vs baseline: 36.5209x; 1.1097x over previous
import functools

import jax
import jax.numpy as jnp
from jax import lax
from jax.experimental import pallas as pl
from jax.experimental.pallas import tpu as pltpu
from jax.experimental.pallas import tpu_sc as plsc

N_NODES = 10000
N_EDGES = 320000
D_FEAT = 128
L = 16
NPAD = 10240
NW = 32
EPW = N_EDGES // NW
CHUNK = 128
NCHUNK = EPW // CHUNK
TAIL = EPW - NCHUNK * CHUNK
ROWS_PER_TILE = NPAD // 16
ZROWS = 128


def _mm_body(x_ref, w_ref, o_ref):
    o_ref[...] = jnp.dot(x_ref[...], w_ref[...],
                         preferred_element_type=jnp.float32).astype(jnp.bfloat16)


def _node_matmul(features, w1p):
    return pl.pallas_call(
        _mm_body,
        grid=(10,),
        in_specs=[
            pl.BlockSpec((N_NODES // 10, D_FEAT), lambda i: (i, 0)),
            pl.BlockSpec((D_FEAT, L), lambda i: (0, 0)),
        ],
        out_specs=pl.BlockSpec((N_NODES // 10, L), lambda i: (i, 0)),
        out_shape=jax.ShapeDtypeStruct((N_NODES, L), jnp.bfloat16),
    )(features, w1p)


def _edge_body(g_hbm, src_hbm, dst_hbm, h1_out, cnt_out,
               src_all, dst_all, rows0, rows1, rows2, rows3, rows4, rows5,
               rowst, zbuf, cnt_priv, h1_sh,
               gsem0, gsem1, gsem2, gsem3, gsem4, gsem5,
               ssem0, ssem1, ssem2, ssem3, ssem4, ssem5):
    c = lax.axis_index("c")
    s = lax.axis_index("s")
    rows_v = (rows0, rows1, rows2, rows3, rows4, rows5)
    gsem = (gsem0, gsem1, gsem2, gsem3, gsem4, gsem5)
    ssem = (ssem0, ssem1, ssem2, ssem3, ssem4, ssem5)

    def _fill_c(i, carry):
        cnt_priv[pl.ds(i * L, L)] = jnp.zeros((L,), jnp.float32)
        return carry
    lax.fori_loop(0, NPAD // L, _fill_c, 0)

    def _fill_z(i, carry):
        zbuf[pl.ds(2 * i, 2), :] = jnp.zeros((2, L), jnp.bfloat16)
        return carry
    lax.fori_loop(0, ZROWS // 2, _fill_z, 0)
    row0 = s * ROWS_PER_TILE
    for j in range(ROWS_PER_TILE // ZROWS):
        pltpu.sync_copy(zbuf, h1_sh.at[pl.ds(row0 + j * ZROWS, ZROWS)])
    plsc.subcore_barrier()

    wid = s * 2 + c
    base = wid * EPW
    pltpu.sync_copy(src_hbm.at[pl.ds(base, EPW)], src_all)
    pltpu.sync_copy(dst_hbm.at[pl.ds(base, EPW)], dst_all)

    def _src_idx(k, n=CHUNK):
        return src_all.at[pl.ds(k * CHUNK, n)]

    def _dst_idx(k, n=CHUNK):
        return dst_all.at[pl.ds(k * CHUNK, n)]

    def _fire_gather(k, slot):
        pltpu.async_copy(g_hbm.at[_src_idx(k)], rows_v[slot], gsem[slot])

    def _wait_gather(k, slot):
        pltpu.make_async_copy(g_hbm.at[_src_idx(k)], rows_v[slot],
                              gsem[slot]).wait()

    def _fire_scatter(k, slot):
        pltpu.async_copy(rows_v[slot], h1_sh.at[_dst_idx(k)], ssem[slot],
                         add=True)

    def _wait_scatter(k, slot):
        pltpu.make_async_copy(rows_v[slot], h1_sh.at[_dst_idx(k)],
                              ssem[slot]).wait()

    iota16 = lax.iota(jnp.int32, L)
    ones16 = jnp.ones((L,), jnp.float32)

    def _histogram(k):
        for j in range(CHUNK // L):
            pos = iota16 + (k * CHUNK + j * L)
            ids = plsc.load_gather(src_all, [pos])
            plsc.addupdate_scatter(cnt_priv, [ids], ones16)

    _fire_gather(0, 0)
    _fire_gather(1, 1)
    _fire_gather(2, 2)

    def _step(k, slot):
        _wait_gather(k, slot)
        _histogram(k)
        _fire_scatter(k, slot)
        nslot = (slot + 3) % 6

        @pl.when(k >= 3)
        def _():
            _wait_scatter(k - 3, nslot)

        @pl.when(k + 3 < NCHUNK)
        def _():
            _fire_gather(k + 3, nslot)

    def _sextet(q, carry):
        for r in range(6):
            _step(6 * q + r, r)
        return carry
    lax.fori_loop(0, NCHUNK // 6, _sextet, 0)

    _wait_scatter(NCHUNK - 3, (NCHUNK - 3) % 6)
    _wait_scatter(NCHUNK - 2, (NCHUNK - 2) % 6)
    _wait_scatter(NCHUNK - 1, (NCHUNK - 1) % 6)

    pltpu.async_copy(g_hbm.at[_src_idx(NCHUNK, TAIL)], rowst, gsem0).wait()
    for j in range(TAIL // L):
        pos = iota16 + (NCHUNK * CHUNK + j * L)
        ids = plsc.load_gather(src_all, [pos])
        plsc.addupdate_scatter(cnt_priv, [ids], ones16)
    pltpu.sync_copy(rowst, h1_sh.at[_dst_idx(NCHUNK, TAIL)], add=True)

    plsc.subcore_barrier()

    out_row = c * NPAD + row0
    pltpu.sync_copy(h1_sh.at[pl.ds(row0, ROWS_PER_TILE)],
                    h1_out.at[pl.ds(out_row, ROWS_PER_TILE)])
    pltpu.sync_copy(cnt_priv, cnt_out.at[wid])


def _edge_pass(g, src, dst):
    mesh = plsc.VectorSubcoreMesh(core_axis_name="c", subcore_axis_name="s",
                                  num_cores=2, num_subcores=16)
    fn = functools.partial(
        pl.kernel,
        mesh=mesh,
        compiler_params=pltpu.CompilerParams(use_tc_tiling_on_sc=False,
                                             needs_layout_passes=False),
        out_type=[
            jax.ShapeDtypeStruct((2 * NPAD, L), jnp.bfloat16),
            jax.ShapeDtypeStruct((NW, NPAD), jnp.float32),
        ],
        scratch_types=[
            pltpu.VMEM((EPW,), jnp.int32),
            pltpu.VMEM((EPW,), jnp.int32),
            pltpu.VMEM((CHUNK, L), jnp.bfloat16),
            pltpu.VMEM((CHUNK, L), jnp.bfloat16),
            pltpu.VMEM((CHUNK, L), jnp.bfloat16),
            pltpu.VMEM((CHUNK, L), jnp.bfloat16),
            pltpu.VMEM((CHUNK, L), jnp.bfloat16),
            pltpu.VMEM((CHUNK, L), jnp.bfloat16),
            pltpu.VMEM((TAIL, L), jnp.bfloat16),
            pltpu.VMEM((ZROWS, L), jnp.bfloat16),
            pltpu.VMEM((NPAD,), jnp.float32),
            pltpu.VMEM_SHARED((NPAD, L), jnp.bfloat16),
            pltpu.SemaphoreType.DMA,
            pltpu.SemaphoreType.DMA,
            pltpu.SemaphoreType.DMA,
            pltpu.SemaphoreType.DMA,
            pltpu.SemaphoreType.DMA,
            pltpu.SemaphoreType.DMA,
            pltpu.SemaphoreType.DMA,
            pltpu.SemaphoreType.DMA,
            pltpu.SemaphoreType.DMA,
            pltpu.SemaphoreType.DMA,
            pltpu.SemaphoreType.DMA,
            pltpu.SemaphoreType.DMA,
        ],
    )(_edge_body)
    return fn(g, src, dst)


def _final_body(h_ref, c_ref, w2_ref, w3_ref, o_ref):
    h = (h_ref[0:NPAD, :].astype(jnp.float32)
         + h_ref[NPAD:2 * NPAD, :].astype(jnp.float32))
    cnt_row = jnp.sum(c_ref[...], axis=0, keepdims=True)
    z = jnp.maximum(h, 0.0)
    sm = jnp.dot(cnt_row, z, preferred_element_type=jnp.float32) * (1.0 / N_NODES)
    v = jnp.dot(sm, w2_ref[...], preferred_element_type=jnp.float32)
    o = jnp.dot(v, w3_ref[...], preferred_element_type=jnp.float32)
    o_ref[...] = 1.0 / (1.0 + jnp.exp(-o))


def _final(h1p, cntp, w2p, w3p):
    return pl.pallas_call(
        _final_body,
        out_shape=jax.ShapeDtypeStruct((1, L), jnp.float32),
    )(h1p, cntp, w2p, w3p)


def kernel(features, edge_index, W1, W2, W3):
    src = edge_index[0].astype(jnp.int32)
    dst = edge_index[1].astype(jnp.int32)
    w1p = jnp.pad(W1.astype(jnp.float32), ((0, 0), (0, L - W1.shape[1])))
    w2p = jnp.pad(W2.astype(jnp.float32), ((0, L - W2.shape[0]), (0, L - W2.shape[1])))
    w3p = jnp.pad(W3.astype(jnp.float32), ((0, L - W3.shape[0]), (0, L - W3.shape[1])))
    g = _node_matmul(features.astype(jnp.float32), w1p)
    h1p, cntp = _edge_pass(g, src, dst)
    full = _final(h1p, cntp, w2p, w3p)
    return full[:, 0:1]
